# baseline (speedup 1.0000x reference)
import functools

import numpy as np
import jax
import jax.numpy as jnp
from jax import lax
from jax.experimental import pallas as pl
from jax.experimental.pallas import tpu as pltpu
from jax.experimental.pallas import tpu_sc as plsc

N = 10000
S = 16
N_PAD = 10240
B_PAD = N_PAD * S
NW = 32
ROWS_PER_W = B_PAD // NW
CHUNK = 128
NCHUNK = ROWS_PER_W // CHUNK
NBUF = 5
GAHEAD = 3

def _elu(x):
    return jnp.where(x > 0, x, jnp.exp(jnp.minimum(x, 0.0)) - 1.0)


def _pack(acc):
    n = acc.shape[-1]
    bits = pltpu.bitcast(acc.astype(jnp.bfloat16).astype(jnp.float32),
                         jnp.uint32)
    return (bits[:, : n // 2] >> 16) | ((bits[:, n // 2:] >> 16) << 16)


def _unpack(xu):
    lo = pltpu.bitcast(xu << 16, jnp.float32).astype(jnp.bfloat16)
    hi = pltpu.bitcast(xu & np.uint32(0xFFFF0000), jnp.float32).astype(jnp.bfloat16)
    return lo, hi


CH_FAST = 60
CH_SLOW = 20
FAST_CORE = 1
IDX_LEN = (16 * CH_FAST + 15 * CH_SLOW + max(CH_FAST, CH_SLOW)) * CHUNK


def _sc_gather(table, idx, n_rows, cols):
    mesh = plsc.VectorSubcoreMesh(core_axis_name="c", subcore_axis_name="s")
    dt = table.dtype
    max_ch = max(CH_FAST, CH_SLOW)

    @functools.partial(
        pl.kernel,
        mesh=mesh,
        out_type=jax.ShapeDtypeStruct((n_rows, cols), dt),
        scratch_types=(
            [pltpu.VMEM((max_ch * CHUNK,), jnp.int32)]
            + [pltpu.VMEM((CHUNK, cols), dt)] * NBUF
            + [pltpu.SemaphoreType.DMA] * (2 * NBUF)
        ),
    )
    def gather_kernel(table_hbm, idx_hbm, out_hbm, idx_v, *rest):
        bufs = rest[:NBUF]
        gsem = rest[NBUF:2 * NBUF]
        ssem = rest[2 * NBUF:]
        c = lax.axis_index("c")
        s = lax.axis_index("s")
        is_fast = c == FAST_CORE
        nch = jnp.where(is_fast, CH_FAST, CH_SLOW)
        base_chunk = jnp.where(
            is_fast, s * CH_FAST, 16 * CH_FAST + s * CH_SLOW)
        base = base_chunk * CHUNK
        pltpu.sync_copy(idx_hbm.at[pl.ds(base, max_ch * CHUNK)], idx_v)

        def fire_gather(i, b):
            pltpu.make_async_copy(
                table_hbm.at[idx_v.at[pl.ds(i * CHUNK, CHUNK)]],
                bufs[b], gsem[b]).start()

        def wait_gather(b):
            pltpu.make_async_copy(
                table_hbm.at[idx_v.at[pl.ds(0, CHUNK)]],
                bufs[b], gsem[b]).wait()

        def fire_store(i, b):
            pltpu.make_async_copy(
                bufs[b], out_hbm.at[pl.ds(base + i * CHUNK, CHUNK)],
                ssem[b]).start()

        def wait_store(b):
            pltpu.make_async_copy(
                bufs[b], out_hbm.at[pl.ds(base, CHUNK)], ssem[b]).wait()

        for b in range(NBUF):
            fire_gather(b, b)
        for b in range(NBUF - GAHEAD):
            wait_gather(b)
            fire_store(b, b)

        def outer(k, carry):
            for b in range(NBUF):
                i = k * NBUF + b
                wait_store(b)
                fire_gather(i, b)
                bp = (b - GAHEAD) % NBUF
                wait_gather(bp)
                fire_store(i - GAHEAD, bp)
            return carry

        lax.fori_loop(1, nch // NBUF, outer, 0)

        for t in range(GAHEAD):
            j = nch - GAHEAD + t
            b = (t - GAHEAD) % NBUF
            wait_gather(b)
            fire_store(j, b)
        for b in range(NBUF):
            wait_store(b)

    return gather_kernel(table, idx)


def _fc0(x, w, b, bm):
    m, k = x.shape
    n = w.shape[1]

    def body(x_ref, w_ref, b_ref, o_ref):
        acc = jnp.dot(x_ref[...], w_ref[...], preferred_element_type=jnp.float32)
        o_ref[...] = _elu(acc + b_ref[...])

    return pl.pallas_call(
        body,
        grid=(m // bm,),
        in_specs=[
            pl.BlockSpec((bm, k), lambda i: (i, 0)),
            pl.BlockSpec((k, n), lambda i: (0, 0)),
            pl.BlockSpec((1, n), lambda i: (0, 0)),
        ],
        out_specs=pl.BlockSpec((bm, n), lambda i: (i, 0)),
        out_shape=jax.ShapeDtypeStruct((m, n), jnp.float32),
    )(x, w, b.reshape(1, n))


def _conv_f32in(g, w, b, bm):
    m, k = g.shape
    n = w.shape[1]

    def body(g_ref, w_ref, b_ref, o_ref):
        acc = jnp.dot(g_ref[...].astype(jnp.bfloat16), w_ref[...],
                      preferred_element_type=jnp.float32)
        o_ref[...] = _pack(_elu(acc + b_ref[...]))

    return pl.pallas_call(
        body,
        grid=(m // bm,),
        in_specs=[
            pl.BlockSpec((bm, k), lambda i: (i, 0)),
            pl.BlockSpec((k, n), lambda i: (0, 0)),
            pl.BlockSpec((1, n), lambda i: (0, 0)),
        ],
        out_specs=pl.BlockSpec((bm, n // 2), lambda i: (i, 0)),
        out_shape=jax.ShapeDtypeStruct((m, n // 2), jnp.uint32),
    )(g, w, b.reshape(1, n))


def _conv(g, wlo, whi, b, bm):
    m, k = g.shape
    n = wlo.shape[1]

    def body(g_ref, wlo_ref, whi_ref, b_ref, o_ref):
        lo, hi = _unpack(g_ref[...])
        acc = jnp.dot(lo, wlo_ref[...], preferred_element_type=jnp.float32)
        acc += jnp.dot(hi, whi_ref[...], preferred_element_type=jnp.float32)
        o_ref[...] = _pack(_elu(acc + b_ref[...]))

    return pl.pallas_call(
        body,
        grid=(m // bm,),
        in_specs=[
            pl.BlockSpec((bm, k), lambda i: (i, 0)),
            pl.BlockSpec((k, n), lambda i: (0, 0)),
            pl.BlockSpec((k, n), lambda i: (0, 0)),
            pl.BlockSpec((1, n), lambda i: (0, 0)),
        ],
        out_specs=pl.BlockSpec((bm, n // 2), lambda i: (i, 0)),
        out_shape=jax.ShapeDtypeStruct((m, n // 2), jnp.uint32),
    )(g, wlo, whi, b.reshape(1, n))


def _head(hp, mass2d, w1lo, w1hi, fc1_b, fc2_W, fc2_b, bm):
    m, k = hp.shape
    n = w1lo.shape[1]
    nc = fc2_W.shape[1]
    grid = m // bm

    def body(h_ref, mass_ref, w1lo_ref, w1hi_ref, b1_ref, w2_ref, b2_ref,
             o_ref, mz_ref, ms_ref):
        i = pl.program_id(0)

        @pl.when(i == 0)
        def _():
            mz_ref[...] = jnp.zeros_like(mz_ref)
            ms_ref[...] = jnp.zeros_like(ms_ref)

        lo, hi = _unpack(h_ref[...])
        z = jnp.dot(lo, w1lo_ref[...], preferred_element_type=jnp.float32)
        z += jnp.dot(hi, w1hi_ref[...], preferred_element_type=jnp.float32)
        z = _elu(z + b1_ref[...])
        mz_ref[...] += jnp.dot(mass_ref[...], z, preferred_element_type=jnp.float32)
        ms_ref[...] = ms_ref[...] + jnp.sum(mass_ref[...])

        @pl.when(i == grid - 1)
        def _():
            pooled = jnp.dot(mz_ref[...], w2_ref[...],
                             preferred_element_type=jnp.float32) / ms_ref[...]
            pooled = pooled + b2_ref[...]
            mx = jnp.max(pooled, axis=-1, keepdims=True)
            sh = pooled - mx
            o_ref[...] = sh - jnp.log(jnp.sum(jnp.exp(sh), axis=-1, keepdims=True))

    return pl.pallas_call(
        body,
        grid=(grid,),
        in_specs=[
            pl.BlockSpec((bm, k), lambda i: (i, 0)),
            pl.BlockSpec((1, bm), lambda i: (0, i)),
            pl.BlockSpec((k, n), lambda i: (0, 0)),
            pl.BlockSpec((k, n), lambda i: (0, 0)),
            pl.BlockSpec((1, n), lambda i: (0, 0)),
            pl.BlockSpec((n, nc), lambda i: (0, 0)),
            pl.BlockSpec((1, nc), lambda i: (0, 0)),
        ],
        out_specs=pl.BlockSpec((1, nc), lambda i: (0, 0)),
        out_shape=jax.ShapeDtypeStruct((1, nc), jnp.float32),
        scratch_shapes=[
            pltpu.VMEM((1, n), jnp.float32),
            pltpu.VMEM((1, 1), jnp.float32),
        ],
    )(hp, mass2d, w1lo, w1hi, fc1_b.reshape(1, n), fc2_W, fc2_b.reshape(1, nc))


def _conv_w_halves(W):
    s, c, o = W.shape
    bf = jnp.bfloat16
    lo = W[:, : c // 2, :].reshape(s * c // 2, o).astype(bf)
    hi = W[:, c // 2:, :].reshape(s * c // 2, o).astype(bf)
    return lo, hi


def kernel(x, indices, mass, fc0_W, fc0_b, conv1_W, conv1_b, conv2_W, conv2_b,
           conv3_W, conv3_b, fc1_W, fc1_b, fc2_W, fc2_b):
    bf = jnp.bfloat16
    idx_flat = jnp.pad(indices.reshape(-1),
                       (0, max(B_PAD, IDX_LEN) - N * S))

    h1 = _fc0(x, fc0_W, fc0_b, 1000)

    s, cin, cout = conv1_W.shape
    g = _sc_gather(h1, idx_flat, B_PAD, cin)
    g = g.reshape(N_PAD, s * cin)
    hp = _conv_f32in(g, conv1_W.reshape(s * cin, cout).astype(bf), conv1_b,
                     1024)

    for W, b in ((conv2_W, conv2_b), (conv3_W, conv3_b)):
        s, cin, cout = W.shape
        g = _sc_gather(hp, idx_flat, B_PAD, cin // 2)
        g = g.reshape(N_PAD, s * cin // 2)
        wlo, whi = _conv_w_halves(W)
        hp = _conv(g, wlo, whi, b, 1024)

    mass2d = jnp.pad(mass, (0, N_PAD - N)).reshape(1, N_PAD)
    k1 = fc1_W.shape[0]
    w1lo = fc1_W[: k1 // 2, :].astype(bf)
    w1hi = fc1_W[k1 // 2:, :].astype(bf)
    out = _head(hp, mass2d, w1lo, w1hi, fc1_b, fc2_W, fc2_b, 1024)
    return out.reshape(-1)


# speedup vs baseline: 1.2146x; 1.2146x over previous
import functools

import numpy as np
import jax
import jax.numpy as jnp
from jax import lax
from jax.experimental import pallas as pl
from jax.experimental.pallas import tpu as pltpu
from jax.experimental.pallas import tpu_sc as plsc

N = 10000
S = 16
N_PAD = 10240
B_PAD = N_PAD * S
NW = 32
ROWS_PER_W = B_PAD // NW
CHUNK = 128
NCHUNK = ROWS_PER_W // CHUNK
NBUF = 5
GAHEAD = 3

def _elu(x):
    return jnp.where(x > 0, x, jnp.exp(jnp.minimum(x, 0.0)) - 1.0)


def _pack(acc):
    n = acc.shape[-1]
    bits = pltpu.bitcast(acc.astype(jnp.bfloat16).astype(jnp.float32),
                         jnp.uint32)
    return (bits[:, : n // 2] >> 16) | ((bits[:, n // 2:] >> 16) << 16)


def _unpack(xu):
    lo = pltpu.bitcast(xu << 16, jnp.float32).astype(jnp.bfloat16)
    hi = pltpu.bitcast(xu & np.uint32(0xFFFF0000), jnp.float32).astype(jnp.bfloat16)
    return lo, hi


def _sc_gather(table, idx, n_rows, cols):
    mesh = plsc.VectorSubcoreMesh(core_axis_name="c", subcore_axis_name="s")
    dt = table.dtype
    nch = n_rows // (NW * CHUNK)
    rows_w = nch * CHUNK

    @functools.partial(
        pl.kernel,
        mesh=mesh,
        out_type=jax.ShapeDtypeStruct((n_rows, cols), dt),
        scratch_types=(
            [pltpu.VMEM((rows_w,), jnp.int32)]
            + [pltpu.VMEM((CHUNK, cols), dt)] * NBUF
            + [pltpu.SemaphoreType.DMA] * (2 * NBUF)
        ),
    )
    def gather_kernel(table_hbm, idx_hbm, out_hbm, idx_v, *rest):
        bufs = rest[:NBUF]
        gsem = rest[NBUF:2 * NBUF]
        ssem = rest[2 * NBUF:]
        wid = lax.axis_index("s") * 2 + lax.axis_index("c")
        base = wid * rows_w
        pltpu.sync_copy(idx_hbm.at[pl.ds(base, rows_w)], idx_v)

        def fire_gather(i, b):
            pltpu.make_async_copy(
                table_hbm.at[idx_v.at[pl.ds(i * CHUNK, CHUNK)]],
                bufs[b], gsem[b]).start()

        def wait_gather(b):
            pltpu.make_async_copy(
                table_hbm.at[idx_v.at[pl.ds(0, CHUNK)]],
                bufs[b], gsem[b]).wait()

        def fire_store(i, b):
            pltpu.make_async_copy(
                bufs[b], out_hbm.at[pl.ds(base + i * CHUNK, CHUNK)],
                ssem[b]).start()

        def wait_store(b):
            pltpu.make_async_copy(
                bufs[b], out_hbm.at[pl.ds(base, CHUNK)], ssem[b]).wait()

        for b in range(NBUF):
            fire_gather(b, b)
        for b in range(NBUF - GAHEAD):
            wait_gather(b)
            fire_store(b, b)

        def outer(k, carry):
            for b in range(NBUF):
                i = k * NBUF + b
                wait_store(b)
                fire_gather(i, b)
                bp = (b - GAHEAD) % NBUF
                wait_gather(bp)
                fire_store(i - GAHEAD, bp)
            return carry

        lax.fori_loop(1, nch // NBUF, outer, 0)

        for j in range(nch - GAHEAD, nch):
            wait_gather(j % NBUF)
            fire_store(j, j % NBUF)
        for b in range(NBUF):
            wait_store(b)

    return gather_kernel(table, idx)


def _fc0(x, w, b, bm):
    m, k = x.shape
    n = w.shape[1]

    def body(x_ref, w_ref, b_ref, o_ref):
        acc = jnp.dot(x_ref[...], w_ref[...], preferred_element_type=jnp.float32)
        o_ref[...] = _elu(acc + b_ref[...])

    return pl.pallas_call(
        body,
        grid=(m // bm,),
        in_specs=[
            pl.BlockSpec((bm, k), lambda i: (i, 0)),
            pl.BlockSpec((k, n), lambda i: (0, 0)),
            pl.BlockSpec((1, n), lambda i: (0, 0)),
        ],
        out_specs=pl.BlockSpec((bm, n), lambda i: (i, 0)),
        out_shape=jax.ShapeDtypeStruct((m, n), jnp.float32),
    )(x, w, b.reshape(1, n))


def _conv_f32in(g, w, b, bm):
    m, k = g.shape
    n = w.shape[1]

    def body(g_ref, w_ref, b_ref, o_ref):
        acc = jnp.dot(g_ref[...].astype(jnp.bfloat16), w_ref[...],
                      preferred_element_type=jnp.float32)
        o_ref[...] = _pack(_elu(acc + b_ref[...]))

    return pl.pallas_call(
        body,
        grid=(m // bm,),
        in_specs=[
            pl.BlockSpec((bm, k), lambda i: (i, 0)),
            pl.BlockSpec((k, n), lambda i: (0, 0)),
            pl.BlockSpec((1, n), lambda i: (0, 0)),
        ],
        out_specs=pl.BlockSpec((bm, n // 2), lambda i: (i, 0)),
        out_shape=jax.ShapeDtypeStruct((m, n // 2), jnp.uint32),
    )(g, w, b.reshape(1, n))


def _conv(g, wlo, whi, b, bm):
    m, k = g.shape
    n = wlo.shape[1]

    def body(g_ref, wlo_ref, whi_ref, b_ref, o_ref):
        lo, hi = _unpack(g_ref[...])
        acc = jnp.dot(lo, wlo_ref[...], preferred_element_type=jnp.float32)
        acc += jnp.dot(hi, whi_ref[...], preferred_element_type=jnp.float32)
        o_ref[...] = _pack(_elu(acc + b_ref[...]))

    return pl.pallas_call(
        body,
        grid=(m // bm,),
        in_specs=[
            pl.BlockSpec((bm, k), lambda i: (i, 0)),
            pl.BlockSpec((k, n), lambda i: (0, 0)),
            pl.BlockSpec((k, n), lambda i: (0, 0)),
            pl.BlockSpec((1, n), lambda i: (0, 0)),
        ],
        out_specs=pl.BlockSpec((bm, n // 2), lambda i: (i, 0)),
        out_shape=jax.ShapeDtypeStruct((m, n // 2), jnp.uint32),
    )(g, wlo, whi, b.reshape(1, n))


def _head(hp, mass2d, w1lo, w1hi, fc1_b, fc2_W, fc2_b, bm):
    m, k = hp.shape
    n = w1lo.shape[1]
    nc = fc2_W.shape[1]
    grid = m // bm

    def body(h_ref, mass_ref, w1lo_ref, w1hi_ref, b1_ref, w2_ref, b2_ref,
             o_ref, mz_ref, ms_ref):
        i = pl.program_id(0)

        @pl.when(i == 0)
        def _():
            mz_ref[...] = jnp.zeros_like(mz_ref)
            ms_ref[...] = jnp.zeros_like(ms_ref)

        lo, hi = _unpack(h_ref[...])
        z = jnp.dot(lo, w1lo_ref[...], preferred_element_type=jnp.float32)
        z += jnp.dot(hi, w1hi_ref[...], preferred_element_type=jnp.float32)
        z = _elu(z + b1_ref[...])
        mz_ref[...] += jnp.dot(mass_ref[...], z, preferred_element_type=jnp.float32)
        ms_ref[...] = ms_ref[...] + jnp.sum(mass_ref[...])

        @pl.when(i == grid - 1)
        def _():
            pooled = jnp.dot(mz_ref[...], w2_ref[...],
                             preferred_element_type=jnp.float32) / ms_ref[...]
            pooled = pooled + b2_ref[...]
            mx = jnp.max(pooled, axis=-1, keepdims=True)
            sh = pooled - mx
            o_ref[...] = sh - jnp.log(jnp.sum(jnp.exp(sh), axis=-1, keepdims=True))

    return pl.pallas_call(
        body,
        grid=(grid,),
        in_specs=[
            pl.BlockSpec((bm, k), lambda i: (i, 0)),
            pl.BlockSpec((1, bm), lambda i: (0, i)),
            pl.BlockSpec((k, n), lambda i: (0, 0)),
            pl.BlockSpec((k, n), lambda i: (0, 0)),
            pl.BlockSpec((1, n), lambda i: (0, 0)),
            pl.BlockSpec((n, nc), lambda i: (0, 0)),
            pl.BlockSpec((1, nc), lambda i: (0, 0)),
        ],
        out_specs=pl.BlockSpec((1, nc), lambda i: (0, 0)),
        out_shape=jax.ShapeDtypeStruct((1, nc), jnp.float32),
        scratch_shapes=[
            pltpu.VMEM((1, n), jnp.float32),
            pltpu.VMEM((1, 1), jnp.float32),
        ],
    )(hp, mass2d, w1lo, w1hi, fc1_b.reshape(1, n), fc2_W, fc2_b.reshape(1, nc))


def _conv_w_halves(W):
    s, c, o = W.shape
    bf = jnp.bfloat16
    lo = W[:, : c // 2, :].reshape(s * c // 2, o).astype(bf)
    hi = W[:, c // 2:, :].reshape(s * c // 2, o).astype(bf)
    return lo, hi


def kernel(x, indices, mass, fc0_W, fc0_b, conv1_W, conv1_b, conv2_W, conv2_b,
           conv3_W, conv3_b, fc1_W, fc1_b, fc2_W, fc2_b):
    bf = jnp.bfloat16
    idx_flat = jnp.pad(indices.reshape(-1), (0, B_PAD - N * S))
    SPLIT = 2
    bh = B_PAD // SPLIT
    nh = N_PAD // SPLIT
    idx_parts = [idx_flat[j * bh:(j + 1) * bh] for j in range(SPLIT)]

    h1 = _fc0(x, fc0_W, fc0_b, 1000)

    s, cin, cout = conv1_W.shape
    w1 = conv1_W.reshape(s * cin, cout).astype(bf)
    gs = [_sc_gather(h1, ip, bh, cin) for ip in idx_parts]
    hs = [_conv_f32in(g.reshape(nh, s * cin), w1, conv1_b, 1024) for g in gs]
    hp = jnp.concatenate(hs)

    for W, b in ((conv2_W, conv2_b), (conv3_W, conv3_b)):
        s, cin, cout = W.shape
        wlo, whi = _conv_w_halves(W)
        gs = [_sc_gather(hp, ip, bh, cin // 2) for ip in idx_parts]
        hs = [_conv(g.reshape(nh, s * cin // 2), wlo, whi, b, 1024)
              for g in gs]
        hp = jnp.concatenate(hs)

    mass2d = jnp.pad(mass, (0, N_PAD - N)).reshape(1, N_PAD)
    k1 = fc1_W.shape[0]
    w1lo = fc1_W[: k1 // 2, :].astype(bf)
    w1hi = fc1_W[k1 // 2:, :].astype(bf)
    out = _head(hp, mass2d, w1lo, w1hi, fc1_b, fc2_W, fc2_b, 1024)
    return out.reshape(-1)


# speedup vs baseline: 1.5758x; 1.2974x over previous
import functools

import numpy as np
import jax
import jax.numpy as jnp
from jax import lax
from jax.experimental import pallas as pl
from jax.experimental.pallas import tpu as pltpu
from jax.experimental.pallas import tpu_sc as plsc

N = 10000
S = 16
N_PAD = 10240
B_PAD = N_PAD * S
NW = 32
ROWS_PER_W = B_PAD // NW
CHUNK = 128
NCHUNK = ROWS_PER_W // CHUNK
NBUF = 5
GAHEAD = 3

def _elu(x):
    return jnp.where(x > 0, x, jnp.exp(jnp.minimum(x, 0.0)) - 1.0)


def _pack(acc):
    n = acc.shape[-1]
    bits = pltpu.bitcast(acc.astype(jnp.bfloat16).astype(jnp.float32),
                         jnp.uint32)
    return (bits[:, : n // 2] >> 16) | ((bits[:, n // 2:] >> 16) << 16)


def _unpack(xu):
    lo = pltpu.bitcast(xu << 16, jnp.float32).astype(jnp.bfloat16)
    hi = pltpu.bitcast(xu & np.uint32(0xFFFF0000), jnp.float32).astype(jnp.bfloat16)
    return lo, hi


def _sc_gather(table, idx, piece, n_pieces, cols):
    mesh = plsc.VectorSubcoreMesh(core_axis_name="c", subcore_axis_name="s")
    dt = table.dtype
    nh = N_PAD // n_pieces
    rows_w = nh // 2
    nch = rows_w // CHUNK

    @functools.partial(
        pl.kernel,
        mesh=mesh,
        out_type=jax.ShapeDtypeStruct((S * nh, cols), dt),
        scratch_types=(
            [pltpu.VMEM((rows_w,), jnp.int32)]
            + [pltpu.VMEM((CHUNK, cols), dt)] * NBUF
            + [pltpu.SemaphoreType.DMA] * (2 * NBUF)
        ),
    )
    def gather_kernel(table_hbm, idx_hbm, out_hbm, idx_v, *rest):
        bufs = rest[:NBUF]
        gsem = rest[NBUF:2 * NBUF]
        ssem = rest[2 * NBUF:]
        s_w = lax.axis_index("s")
        q = lax.axis_index("c")
        idx_base = s_w * N_PAD + piece * nh + q * rows_w
        out_base = s_w * nh + q * rows_w
        pltpu.sync_copy(idx_hbm.at[pl.ds(idx_base, rows_w)], idx_v)

        def fire_gather(i, b):
            pltpu.make_async_copy(
                table_hbm.at[idx_v.at[pl.ds(i * CHUNK, CHUNK)]],
                bufs[b], gsem[b]).start()

        def wait_gather(b):
            pltpu.make_async_copy(
                table_hbm.at[idx_v.at[pl.ds(0, CHUNK)]],
                bufs[b], gsem[b]).wait()

        def fire_store(i, b):
            pltpu.make_async_copy(
                bufs[b], out_hbm.at[pl.ds(out_base + i * CHUNK, CHUNK)],
                ssem[b]).start()

        def wait_store(b):
            pltpu.make_async_copy(
                bufs[b], out_hbm.at[pl.ds(out_base, CHUNK)], ssem[b]).wait()

        for b in range(NBUF):
            fire_gather(b, b)
        for b in range(NBUF - GAHEAD):
            wait_gather(b)
            fire_store(b, b)

        def outer(k, carry):
            for b in range(NBUF):
                i = k * NBUF + b
                wait_store(b)
                fire_gather(i, b)
                bp = (b - GAHEAD) % NBUF
                wait_gather(bp)
                fire_store(i - GAHEAD, bp)
            return carry

        lax.fori_loop(1, nch // NBUF, outer, 0)

        for j in range(nch - GAHEAD, nch):
            wait_gather(j % NBUF)
            fire_store(j, j % NBUF)
        for b in range(NBUF):
            wait_store(b)

    return gather_kernel(table, idx)


def _fc0(x, w, b, bm):
    m, k = x.shape
    n = w.shape[1]

    def body(x_ref, w_ref, b_ref, o_ref):
        acc = jnp.dot(x_ref[...], w_ref[...], preferred_element_type=jnp.float32)
        o_ref[...] = _elu(acc + b_ref[...])

    return pl.pallas_call(
        body,
        grid=(m // bm,),
        in_specs=[
            pl.BlockSpec((bm, k), lambda i: (i, 0)),
            pl.BlockSpec((k, n), lambda i: (0, 0)),
            pl.BlockSpec((1, n), lambda i: (0, 0)),
        ],
        out_specs=pl.BlockSpec((bm, n), lambda i: (i, 0)),
        out_shape=jax.ShapeDtypeStruct((m, n), jnp.float32),
    )(x, w, b.reshape(1, n))


def _conv_f32in(g3, w3, b, bm):
    _, m, k = g3.shape
    n = w3.shape[2]

    def body(g_ref, w_ref, b_ref, o_ref):
        acc = b_ref[...].astype(jnp.float32) * jnp.ones((bm, 1), jnp.float32)
        for s in range(S):
            acc += jnp.dot(g_ref[s].astype(jnp.bfloat16), w_ref[s],
                           preferred_element_type=jnp.float32)
        o_ref[...] = _pack(_elu(acc))

    return pl.pallas_call(
        body,
        grid=(m // bm,),
        in_specs=[
            pl.BlockSpec((S, bm, k), lambda i: (0, i, 0)),
            pl.BlockSpec((S, k, n), lambda i: (0, 0, 0)),
            pl.BlockSpec((1, n), lambda i: (0, 0)),
        ],
        out_specs=pl.BlockSpec((bm, n // 2), lambda i: (i, 0)),
        out_shape=jax.ShapeDtypeStruct((m, n // 2), jnp.uint32),
    )(g3, w3, b.reshape(1, n))


def _conv(g3, wlo3, whi3, b, bm):
    _, m, k = g3.shape
    n = wlo3.shape[2]

    def body(g_ref, wlo_ref, whi_ref, b_ref, o_ref):
        acc = b_ref[...].astype(jnp.float32) * jnp.ones((bm, 1), jnp.float32)
        for s in range(S):
            lo, hi = _unpack(g_ref[s])
            acc += jnp.dot(lo, wlo_ref[s], preferred_element_type=jnp.float32)
            acc += jnp.dot(hi, whi_ref[s], preferred_element_type=jnp.float32)
        o_ref[...] = _pack(_elu(acc))

    return pl.pallas_call(
        body,
        grid=(m // bm,),
        in_specs=[
            pl.BlockSpec((S, bm, k), lambda i: (0, i, 0)),
            pl.BlockSpec((S, k, n), lambda i: (0, 0, 0)),
            pl.BlockSpec((S, k, n), lambda i: (0, 0, 0)),
            pl.BlockSpec((1, n), lambda i: (0, 0)),
        ],
        out_specs=pl.BlockSpec((bm, n // 2), lambda i: (i, 0)),
        out_shape=jax.ShapeDtypeStruct((m, n // 2), jnp.uint32),
    )(g3, wlo3, whi3, b.reshape(1, n))


def _head(hp, mass2d, w1lo, w1hi, fc1_b, fc2_W, fc2_b, bm):
    m, k = hp.shape
    n = w1lo.shape[1]
    nc = fc2_W.shape[1]
    grid = m // bm

    def body(h_ref, mass_ref, w1lo_ref, w1hi_ref, b1_ref, w2_ref, b2_ref,
             o_ref, mz_ref, ms_ref):
        i = pl.program_id(0)

        @pl.when(i == 0)
        def _():
            mz_ref[...] = jnp.zeros_like(mz_ref)
            ms_ref[...] = jnp.zeros_like(ms_ref)

        lo, hi = _unpack(h_ref[...])
        z = jnp.dot(lo, w1lo_ref[...], preferred_element_type=jnp.float32)
        z += jnp.dot(hi, w1hi_ref[...], preferred_element_type=jnp.float32)
        z = _elu(z + b1_ref[...])
        mz_ref[...] += jnp.dot(mass_ref[...], z, preferred_element_type=jnp.float32)
        ms_ref[...] = ms_ref[...] + jnp.sum(mass_ref[...])

        @pl.when(i == grid - 1)
        def _():
            pooled = jnp.dot(mz_ref[...], w2_ref[...],
                             preferred_element_type=jnp.float32) / ms_ref[...]
            pooled = pooled + b2_ref[...]
            mx = jnp.max(pooled, axis=-1, keepdims=True)
            sh = pooled - mx
            o_ref[...] = sh - jnp.log(jnp.sum(jnp.exp(sh), axis=-1, keepdims=True))

    return pl.pallas_call(
        body,
        grid=(grid,),
        in_specs=[
            pl.BlockSpec((bm, k), lambda i: (i, 0)),
            pl.BlockSpec((1, bm), lambda i: (0, i)),
            pl.BlockSpec((k, n), lambda i: (0, 0)),
            pl.BlockSpec((k, n), lambda i: (0, 0)),
            pl.BlockSpec((1, n), lambda i: (0, 0)),
            pl.BlockSpec((n, nc), lambda i: (0, 0)),
            pl.BlockSpec((1, nc), lambda i: (0, 0)),
        ],
        out_specs=pl.BlockSpec((1, nc), lambda i: (0, 0)),
        out_shape=jax.ShapeDtypeStruct((1, nc), jnp.float32),
        scratch_shapes=[
            pltpu.VMEM((1, n), jnp.float32),
            pltpu.VMEM((1, 1), jnp.float32),
        ],
    )(hp, mass2d, w1lo, w1hi, fc1_b.reshape(1, n), fc2_W, fc2_b.reshape(1, nc))


def _conv_w_halves(W):
    c = W.shape[1]
    bf = jnp.bfloat16
    return W[:, : c // 2, :].astype(bf), W[:, c // 2:, :].astype(bf)


def kernel(x, indices, mass, fc0_W, fc0_b, conv1_W, conv1_b, conv2_W, conv2_b,
           conv3_W, conv3_b, fc1_W, fc1_b, fc2_W, fc2_b):
    bf = jnp.bfloat16
    idx_sm = jnp.pad(indices, ((0, N_PAD - N), (0, 0))).T.reshape(-1)
    P = 2
    nh = N_PAD // P

    h1 = _fc0(x, fc0_W, fc0_b, 1000)

    cin = conv1_W.shape[1]
    w13 = conv1_W.astype(bf)
    gs = [_sc_gather(h1, idx_sm, p, P, cin) for p in range(P)]
    hs = [_conv_f32in(g.reshape(S, nh, cin), w13, conv1_b, 1024) for g in gs]
    hp = jnp.concatenate(hs)

    for W, b in ((conv2_W, conv2_b), (conv3_W, conv3_b)):
        cin = W.shape[1]
        wlo, whi = _conv_w_halves(W)
        gs = [_sc_gather(hp, idx_sm, p, P, cin // 2) for p in range(P)]
        hs = [_conv(g.reshape(S, nh, cin // 2), wlo, whi, b, 1024)
              for g in gs]
        hp = jnp.concatenate(hs)

    mass2d = jnp.pad(mass, (0, N_PAD - N)).reshape(1, N_PAD)
    k1 = fc1_W.shape[0]
    w1lo = fc1_W[: k1 // 2, :].astype(bf)
    w1hi = fc1_W[k1 // 2:, :].astype(bf)
    out = _head(hp, mass2d, w1lo, w1hi, fc1_b, fc2_W, fc2_b, 1024)
    return out.reshape(-1)
